# Initial kernel scaffold; baseline (speedup 1.0000x reference)
import jax
import jax.numpy as jnp
from jax.experimental import pallas as pl


def kernel(x, edge_index, W1, b1, W2, b2):
    raise NotImplementedError("write your pallas kernel here")


# speedup vs baseline: 11.4568x; 11.4568x over previous
import functools

import jax
import jax.numpy as jnp
from jax import lax
from jax.experimental import pallas as pl
from jax.experimental.pallas import tpu as pltpu
from jax.experimental.pallas import tpu_sc as plsc

N = 10000
E = 160000
D_IN = 256
D_H = 512
D_OUT = 256

NC = 2
NS = 16
L = 16

NPAD = 10240
COLS_PER_TILE = NPAD // NS

BATCH = 128
NB = -(-(E + N) // (NS * BATCH))
E2 = NS * NB * BATCH
ACC_ROWS = NPAD
ROWS_PER_TILE = N // NS
RED_BATCH = 25
IDXCHUNK = 12
NCHUNK = NB // IDXCHUNK


def _rsqrt_newton(d):
  xi = plsc.bitcast(d, jnp.int32)
  i = jnp.int32(0x5F3759DF) - lax.shift_right_logical(xi, 1)
  r = plsc.bitcast(i, jnp.float32)
  for _ in range(3):
    r = r * (1.5 - 0.5 * d * r * r)
  return r


EPT_A = E // NS


def _stats_body(src_hbm, dst_hbm, dis_hbm, w_hbm,
                part_v, idx_v, idx2_v, disfull_v, red_v, chunk_v,
                stage_sh, dis_sh):
  c = lax.axis_index("c")
  s = lax.axis_index("s")

  @pl.when(c == 0)
  def _():
    zeros16 = jnp.zeros((L,), jnp.float32)
    ones16 = jnp.ones((L,), jnp.float32)

    def zero_part(i, _):
      part_v[pl.ds(i * L, L)] = zeros16
      return 0
    lax.fori_loop(0, NPAD // L, zero_part, 0)

    pltpu.sync_copy(dst_hbm.at[pl.ds(s * EPT_A, EPT_A)], idx_v)

    def deg_step(i, _):
      d_idx = idx_v[pl.ds(i * L, L)]
      plsc.addupdate_scatter(part_v, [d_idx], ones16)
      return 0
    lax.fori_loop(0, EPT_A // L, deg_step, 0)

    pltpu.sync_copy(part_v, stage_sh.at[s])
    plsc.subcore_barrier()

    col0 = s * COLS_PER_TILE
    pltpu.sync_copy(stage_sh.at[:, pl.ds(col0, COLS_PER_TILE)], red_v)

    def dis_step(k, _):
      acc = red_v[0, pl.ds(k * L, L)]
      for t in range(1, NS):
        acc = acc + red_v[t, pl.ds(k * L, L)]
      chunk_v[pl.ds(k * L, L)] = _rsqrt_newton(acc + 1.0)
      return 0
    lax.fori_loop(0, COLS_PER_TILE // L, dis_step, 0)

    pltpu.sync_copy(chunk_v, dis_hbm.at[pl.ds(col0, COLS_PER_TILE)])
    pltpu.sync_copy(chunk_v, dis_sh.at[pl.ds(col0, COLS_PER_TILE)])
    plsc.subcore_barrier()

    pltpu.sync_copy(dis_sh, disfull_v)
    pltpu.sync_copy(src_hbm.at[pl.ds(s * EPT_A, EPT_A)], idx2_v)

    lax.fori_loop(0, NPAD // L, zero_part, 0)

    def t_step(i, _):
      d_idx = idx_v[pl.ds(i * L, L)]
      s_idx = idx2_v[pl.ds(i * L, L)]
      dvals = plsc.load_gather(disfull_v, [d_idx])
      plsc.addupdate_scatter(part_v, [s_idx], dvals)
      return 0
    lax.fori_loop(0, EPT_A // L, t_step, 0)

    pltpu.sync_copy(part_v, stage_sh.at[s])
    plsc.subcore_barrier()

    pltpu.sync_copy(stage_sh.at[:, pl.ds(col0, COLS_PER_TILE)], red_v)

    def w_step(k, _):
      acc = red_v[0, pl.ds(k * L, L)]
      for t in range(1, NS):
        acc = acc + red_v[t, pl.ds(k * L, L)]
      dv = disfull_v[pl.ds(col0 + k * L, L)]
      chunk_v[pl.ds(k * L, L)] = dv * (acc + dv)
      return 0
    lax.fori_loop(0, COLS_PER_TILE // L, w_step, 0)

    pltpu.sync_copy(chunk_v, w_hbm.at[pl.ds(col0, COLS_PER_TILE)])


@functools.partial(jax.jit, static_argnames=())
def _sc_stats(src, dst):
  mesh = plsc.VectorSubcoreMesh(core_axis_name="c", subcore_axis_name="s")
  f = pl.kernel(
      _stats_body,
      out_type=(
          jax.ShapeDtypeStruct((NPAD,), jnp.float32),
          jax.ShapeDtypeStruct((NPAD,), jnp.float32),
      ),
      mesh=mesh,
      scratch_types=[
          pltpu.VMEM((NPAD,), jnp.float32),
          pltpu.VMEM((EPT_A,), jnp.int32),
          pltpu.VMEM((EPT_A,), jnp.int32),
          pltpu.VMEM((NPAD,), jnp.float32),
          pltpu.VMEM((NS, COLS_PER_TILE), jnp.float32),
          pltpu.VMEM((COLS_PER_TILE,), jnp.float32),
          pltpu.VMEM_SHARED((NS, NPAD), jnp.float32),
          pltpu.VMEM_SHARED((NPAD,), jnp.float32),
      ],
      compiler_params=pltpu.CompilerParams(use_tc_tiling_on_sc=False, needs_layout_passes=False),
  )
  return f(src, dst)


BM = 400


def _mm_body(x_ref, w_ref, dis_ref, out_ref):
  out_ref[...] = dis_ref[...] * jnp.dot(
      x_ref[...], w_ref[...], preferred_element_type=jnp.float32)


def _tc_scaled_mm(x, W1, dis2):
  return pl.pallas_call(
      _mm_body,
      out_shape=jax.ShapeDtypeStruct((N, D_H), jnp.float32),
      grid=(N // BM,),
      in_specs=[
          pl.BlockSpec((BM, D_IN), lambda i: (i, 0)),
          pl.BlockSpec((D_IN, D_H), lambda i: (0, 0)),
          pl.BlockSpec((BM, 1), lambda i: (i, 0)),
      ],
      out_specs=pl.BlockSpec((BM, D_H), lambda i: (i, 0)),
  )(x, W1, dis2)


NSLAB = D_H // BATCH
PASSES = NSLAB // NC


def _msg_body(y2_hbm, gidx_hbm, didx_hbm, dis_hbm, w_hbm, b1_hbm, out_hbm,
              gidx_v, didx_v, buf0, buf1, zbuf, redbuf,
              dis_v, w_v, b1_v, s_v, sem0, sem1, sem2, sem3, acc_sh):
  c = lax.axis_index("c")
  s = lax.axis_index("s")
  zeros16 = jnp.zeros((L,), jnp.float32)

  def zb(i, _):
    for f in range(BATCH // L):
      zbuf[i, pl.ds(f * L, L)] = zeros16
    return 0
  lax.fori_loop(0, 16, zb, 0)

  row0 = s * ROWS_PER_TILE
  a0 = (row0 // 8) * 8
  off = row0 - a0
  pltpu.sync_copy(dis_hbm.at[pl.ds(a0, COLS_PER_TILE)], dis_v)
  pltpu.sync_copy(w_hbm.at[pl.ds(a0, COLS_PER_TILE)], w_v)

  for q in range(PASSES):
    p = c * PASSES + q

    for z in range(COLS_PER_TILE // 16):
      pltpu.sync_copy(zbuf, acc_sh.at[pl.ds(s * COLS_PER_TILE + z * 16, 16)])

    pltpu.sync_copy(b1_hbm.at[pl.ds(p * BATCH, BATCH)], b1_v)
    plsc.subcore_barrier()

    def fire_g(j, buf, sem):
      pltpu.async_copy(y2_hbm.at[gidx_v.at[j]], buf, sem)

    def wait_g_fire_s(j, buf, gsem, ssem):
      pltpu.make_async_copy(y2_hbm.at[gidx_v.at[j]], buf, gsem).wait()
      pltpu.async_copy(buf, acc_sh.at[didx_v.at[j]], ssem, add=True)

    def wait_s(j, buf, ssem):
      pltpu.make_async_copy(buf, acc_sh.at[didx_v.at[j]], ssem).wait()

    def chunk_step(ic, _):
      j0 = s * NB + ic * IDXCHUNK
      pltpu.sync_copy(gidx_hbm.at[p, pl.ds(j0, IDXCHUNK)], gidx_v)
      pltpu.sync_copy(didx_hbm.at[pl.ds(j0, IDXCHUNK)], didx_v)
      fire_g(0, buf0, sem0)
      fire_g(1, buf1, sem1)

      def edge_step(t, _):
        j = t * 2
        wait_g_fire_s(j, buf0, sem0, sem2)
        wait_g_fire_s(j + 1, buf1, sem1, sem3)
        wait_s(j, buf0, sem2)

        @pl.when(j + 2 < IDXCHUNK)
        def _():
          fire_g(j + 2, buf0, sem0)

        wait_s(j + 1, buf1, sem3)

        @pl.when(j + 3 < IDXCHUNK)
        def _():
          fire_g(j + 3, buf1, sem1)

        return 0
      lax.fori_loop(0, IDXCHUNK // 2, edge_step, 0)
      return 0
    lax.fori_loop(0, NCHUNK, chunk_step, 0)

    plsc.subcore_barrier()

    b1r = tuple(b1_v[pl.ds(f * L, L)] for f in range(BATCH // L))

    def red_step(k, sacc):
      r0 = row0 + k * RED_BATCH
      pltpu.sync_copy(acc_sh.at[pl.ds(r0, RED_BATCH)], redbuf)

      def row_step(r, sacc):
        nvec = jnp.full((L,), off + k * RED_BATCH + r, jnp.int32)
        dval = plsc.load_gather(dis_v, [nvec])
        wval = plsc.load_gather(w_v, [nvec])
        return tuple(
            sacc[f] + wval * jnp.maximum(
                dval * redbuf[r, pl.ds(f * L, L)] + b1r[f], 0.0)
            for f in range(BATCH // L))
      return lax.fori_loop(0, RED_BATCH, row_step, sacc)
    sacc = lax.fori_loop(0, ROWS_PER_TILE // RED_BATCH, red_step,
                         tuple(zeros16 for _ in range(BATCH // L)))
    for f in range(BATCH // L):
      s_v[pl.ds(f * L, L)] = sacc[f]

    pltpu.sync_copy(s_v, out_hbm.at[p * NS + s])
    plsc.subcore_barrier()


def _sc_message(y2, gidx4, didx, dis, w, b1):
  mesh = plsc.VectorSubcoreMesh(core_axis_name="c", subcore_axis_name="s")
  f = pl.kernel(
      _msg_body,
      out_type=jax.ShapeDtypeStruct((NSLAB * NS, BATCH), jnp.float32),
      mesh=mesh,
      scratch_types=[
          pltpu.VMEM((IDXCHUNK, BATCH), jnp.int32),
          pltpu.VMEM((IDXCHUNK, BATCH), jnp.int32),
          pltpu.VMEM((BATCH, BATCH), jnp.float32),
          pltpu.VMEM((BATCH, BATCH), jnp.float32),
          pltpu.VMEM((16, BATCH), jnp.float32),
          pltpu.VMEM((RED_BATCH, BATCH), jnp.float32),
          pltpu.VMEM((COLS_PER_TILE,), jnp.float32),
          pltpu.VMEM((COLS_PER_TILE,), jnp.float32),
          pltpu.VMEM((BATCH,), jnp.float32),
          pltpu.VMEM((BATCH,), jnp.float32),
          pltpu.SemaphoreType.DMA,
          pltpu.SemaphoreType.DMA,
          pltpu.SemaphoreType.DMA,
          pltpu.SemaphoreType.DMA,
          pltpu.VMEM_SHARED((ACC_ROWS, BATCH), jnp.float32),
      ],
      compiler_params=pltpu.CompilerParams(use_tc_tiling_on_sc=False, needs_layout_passes=False),
  )
  return f(y2, gidx4, didx, dis, w, b1)


def _final_body(s_ref, w2_ref, b2_ref, out_ref):
  acc = jnp.zeros((1, D_OUT), jnp.float32)
  for p in range(NSLAB):
    m = jnp.sum(s_ref[p], axis=0, keepdims=True)
    acc = acc + jnp.dot(m, w2_ref[p], preferred_element_type=jnp.float32)
  out_ref[...] = acc * (1.0 / N) + b2_ref[...]


def _tc_final(s_parts, W2r, b2r):
  return pl.pallas_call(
      _final_body,
      out_shape=jax.ShapeDtypeStruct((1, D_OUT), jnp.float32),
  )(s_parts, W2r, b2r)


def kernel(x, edge_index, W1, b1, W2, b2):
  src = edge_index[0]
  dst = edge_index[1]

  dis_pad, w_pad = _sc_stats(src, dst)
  dis = dis_pad[:N]

  y = _tc_scaled_mm(x, W1, dis.reshape(N, 1))
  y2 = y.reshape(N * NSLAB, BATCH)

  loop = jnp.arange(N, dtype=jnp.int32)
  pad = E2 - (E + N)
  src2 = jnp.concatenate([src, loop, jnp.zeros((pad,), jnp.int32)])
  dst2 = jnp.concatenate([dst, loop, jnp.full((pad,), N, jnp.int32)])
  gidx4 = (src2[None, :] * NSLAB
           + jnp.arange(NSLAB, dtype=jnp.int32)[:, None]).reshape(
               NSLAB, NS * NB, BATCH)
  didx = dst2.reshape(NS * NB, BATCH)

  s_parts = _sc_message(y2, gidx4, didx, dis, w_pad[:N], b1)

  out = _tc_final(s_parts.reshape(NSLAB, NS, BATCH),
                  W2.reshape(NSLAB, BATCH, D_OUT),
                  b2.reshape(1, D_OUT))
  return out.reshape(D_OUT)


# speedup vs baseline: 18.4661x; 1.6118x over previous
import functools

import jax
import jax.numpy as jnp
from jax import lax
from jax.experimental import pallas as pl
from jax.experimental.pallas import tpu as pltpu
from jax.experimental.pallas import tpu_sc as plsc

N = 10000
E = 160000
D_IN = 256
D_H = 512
D_OUT = 256

NC = 2
NS = 16
L = 16

NPAD = 10240
COLS_PER_TILE = NPAD // NS

BATCH = 128
NB = -(-(E + N) // (NS * BATCH))
E2 = NS * NB * BATCH
ACC_ROWS = NPAD
ROWS_PER_TILE = N // NS
RED_BATCH = 25
IDXCHUNK = 12
NCHUNK = NB // IDXCHUNK


def _rsqrt_newton(d):
  xi = plsc.bitcast(d, jnp.int32)
  i = jnp.int32(0x5F3759DF) - lax.shift_right_logical(xi, 1)
  r = plsc.bitcast(i, jnp.float32)
  for _ in range(3):
    r = r * (1.5 - 0.5 * d * r * r)
  return r


EPT_A = E // NS


def _stats_body(src_hbm, dst_hbm, dis_hbm, w_hbm,
                part_v, idx_v, idx2_v, disfull_v, red_v, chunk_v,
                stage_sh, dis_sh):
  c = lax.axis_index("c")
  s = lax.axis_index("s")

  @pl.when(c == 0)
  def _():
    zeros16 = jnp.zeros((L,), jnp.float32)
    ones16 = jnp.ones((L,), jnp.float32)

    def zero_part(i, _):
      part_v[pl.ds(i * L, L)] = zeros16
      return 0
    lax.fori_loop(0, NPAD // L, zero_part, 0)

    pltpu.sync_copy(dst_hbm.at[pl.ds(s * EPT_A, EPT_A)], idx_v)

    def deg_step(i, _):
      d_idx = idx_v[pl.ds(i * L, L)]
      plsc.addupdate_scatter(part_v, [d_idx], ones16)
      return 0
    lax.fori_loop(0, EPT_A // L, deg_step, 0)

    pltpu.sync_copy(part_v, stage_sh.at[s])
    plsc.subcore_barrier()

    col0 = s * COLS_PER_TILE
    pltpu.sync_copy(stage_sh.at[:, pl.ds(col0, COLS_PER_TILE)], red_v)

    def dis_step(k, _):
      acc = red_v[0, pl.ds(k * L, L)]
      for t in range(1, NS):
        acc = acc + red_v[t, pl.ds(k * L, L)]
      chunk_v[pl.ds(k * L, L)] = _rsqrt_newton(acc + 1.0)
      return 0
    lax.fori_loop(0, COLS_PER_TILE // L, dis_step, 0)

    pltpu.sync_copy(chunk_v, dis_hbm.at[pl.ds(col0, COLS_PER_TILE)])
    pltpu.sync_copy(chunk_v, dis_sh.at[pl.ds(col0, COLS_PER_TILE)])
    plsc.subcore_barrier()

    pltpu.sync_copy(dis_sh, disfull_v)
    pltpu.sync_copy(src_hbm.at[pl.ds(s * EPT_A, EPT_A)], idx2_v)

    lax.fori_loop(0, NPAD // L, zero_part, 0)

    def t_step(i, _):
      d_idx = idx_v[pl.ds(i * L, L)]
      s_idx = idx2_v[pl.ds(i * L, L)]
      dvals = plsc.load_gather(disfull_v, [d_idx])
      plsc.addupdate_scatter(part_v, [s_idx], dvals)
      return 0
    lax.fori_loop(0, EPT_A // L, t_step, 0)

    pltpu.sync_copy(part_v, stage_sh.at[s])
    plsc.subcore_barrier()

    pltpu.sync_copy(stage_sh.at[:, pl.ds(col0, COLS_PER_TILE)], red_v)

    def w_step(k, _):
      acc = red_v[0, pl.ds(k * L, L)]
      for t in range(1, NS):
        acc = acc + red_v[t, pl.ds(k * L, L)]
      dv = disfull_v[pl.ds(col0 + k * L, L)]
      chunk_v[pl.ds(k * L, L)] = dv * (acc + dv)
      return 0
    lax.fori_loop(0, COLS_PER_TILE // L, w_step, 0)

    pltpu.sync_copy(chunk_v, w_hbm.at[pl.ds(col0, COLS_PER_TILE)])


@functools.partial(jax.jit, static_argnames=())
def _sc_stats(src, dst):
  mesh = plsc.VectorSubcoreMesh(core_axis_name="c", subcore_axis_name="s")
  f = pl.kernel(
      _stats_body,
      out_type=(
          jax.ShapeDtypeStruct((NPAD,), jnp.float32),
          jax.ShapeDtypeStruct((NPAD,), jnp.float32),
      ),
      mesh=mesh,
      scratch_types=[
          pltpu.VMEM((NPAD,), jnp.float32),
          pltpu.VMEM((EPT_A,), jnp.int32),
          pltpu.VMEM((EPT_A,), jnp.int32),
          pltpu.VMEM((NPAD,), jnp.float32),
          pltpu.VMEM((NS, COLS_PER_TILE), jnp.float32),
          pltpu.VMEM((COLS_PER_TILE,), jnp.float32),
          pltpu.VMEM_SHARED((NS, NPAD), jnp.float32),
          pltpu.VMEM_SHARED((NPAD,), jnp.float32),
      ],
      compiler_params=pltpu.CompilerParams(use_tc_tiling_on_sc=False, needs_layout_passes=False),
  )
  return f(src, dst)


BM = 400


def _mm_body(x_ref, w_ref, dis_ref, out_ref):
  out_ref[...] = (dis_ref[...] * jnp.dot(
      x_ref[...], w_ref[...],
      preferred_element_type=jnp.float32)).astype(jnp.bfloat16)


def _tc_scaled_mm(x, W1, dis2):
  return pl.pallas_call(
      _mm_body,
      out_shape=jax.ShapeDtypeStruct((N, D_H), jnp.bfloat16),
      grid=(N // BM,),
      in_specs=[
          pl.BlockSpec((BM, D_IN), lambda i: (i, 0)),
          pl.BlockSpec((D_IN, D_H), lambda i: (0, 0)),
          pl.BlockSpec((BM, 1), lambda i: (i, 0)),
      ],
      out_specs=pl.BlockSpec((BM, D_H), lambda i: (i, 0)),
  )(x, W1, dis2)


SLAB = D_H // NC


def _msg_body(y2_hbm, gidx_hbm, didx_hbm, out_hbm,
              gidx_v, didx_v, buf0, buf1, zbuf,
              sem0, sem1, sem2, sem3, acc_sh):
  c = lax.axis_index("c")
  s = lax.axis_index("s")
  zeros32 = jnp.zeros((2 * L,), jnp.bfloat16)

  def zb(i, _):
    for f in range(SLAB // (2 * L)):
      zbuf[i, pl.ds(f * 2 * L, 2 * L)] = zeros32
    return 0
  lax.fori_loop(0, 16, zb, 0)

  for z in range(COLS_PER_TILE // 16):
    pltpu.sync_copy(zbuf, acc_sh.at[pl.ds(s * COLS_PER_TILE + z * 16, 16)])
  plsc.subcore_barrier()

  def fire_g(j, buf, sem):
    pltpu.async_copy(y2_hbm.at[gidx_v.at[j]], buf, sem)

  def wait_g_fire_s(j, buf, gsem, ssem):
    pltpu.make_async_copy(y2_hbm.at[gidx_v.at[j]], buf, gsem).wait()
    pltpu.async_copy(buf, acc_sh.at[didx_v.at[j]], ssem, add=True)

  def wait_s(j, buf, ssem):
    pltpu.make_async_copy(buf, acc_sh.at[didx_v.at[j]], ssem).wait()

  def chunk_step(ic, _):
    j0 = s * NB + ic * IDXCHUNK
    pltpu.sync_copy(gidx_hbm.at[c, pl.ds(j0, IDXCHUNK)], gidx_v)
    pltpu.sync_copy(didx_hbm.at[pl.ds(j0, IDXCHUNK)], didx_v)
    fire_g(0, buf0, sem0)
    fire_g(1, buf1, sem1)

    def edge_step(t, _):
      j = t * 2
      wait_g_fire_s(j, buf0, sem0, sem2)
      wait_g_fire_s(j + 1, buf1, sem1, sem3)
      wait_s(j, buf0, sem2)

      @pl.when(j + 2 < IDXCHUNK)
      def _():
        fire_g(j + 2, buf0, sem0)

      wait_s(j + 1, buf1, sem3)

      @pl.when(j + 3 < IDXCHUNK)
      def _():
        fire_g(j + 3, buf1, sem1)

      return 0
    lax.fori_loop(0, IDXCHUNK // 2, edge_step, 0)
    return 0
  lax.fori_loop(0, NCHUNK, chunk_step, 0)

  plsc.subcore_barrier()

  pltpu.sync_copy(acc_sh.at[pl.ds(s * COLS_PER_TILE, COLS_PER_TILE)],
                  out_hbm.at[c, pl.ds(s * COLS_PER_TILE, COLS_PER_TILE)])


def _sc_message(y2, gidx2, didx):
  mesh = plsc.VectorSubcoreMesh(core_axis_name="c", subcore_axis_name="s")
  f = pl.kernel(
      _msg_body,
      out_type=jax.ShapeDtypeStruct((NC, NPAD, SLAB), jnp.bfloat16),
      mesh=mesh,
      scratch_types=[
          pltpu.VMEM((IDXCHUNK, BATCH), jnp.int32),
          pltpu.VMEM((IDXCHUNK, BATCH), jnp.int32),
          pltpu.VMEM((BATCH, SLAB), jnp.bfloat16),
          pltpu.VMEM((BATCH, SLAB), jnp.bfloat16),
          pltpu.VMEM((16, SLAB), jnp.bfloat16),
          pltpu.SemaphoreType.DMA,
          pltpu.SemaphoreType.DMA,
          pltpu.SemaphoreType.DMA,
          pltpu.SemaphoreType.DMA,
          pltpu.VMEM_SHARED((ACC_ROWS, SLAB), jnp.bfloat16),
      ],
      compiler_params=pltpu.CompilerParams(use_tc_tiling_on_sc=False, needs_layout_passes=False),
  )
  return f(y2, gidx2, didx)


BMD = 400


def _red_body(acc_ref, dis_ref, w_ref, b1_ref, w2_ref, b2_ref, out_ref, s_scr):
  i = pl.program_id(0)

  @pl.when(i == 0)
  def _():
    s_scr[...] = jnp.zeros_like(s_scr)

  dis = dis_ref[...]
  wv = w_ref[...]
  for p in range(NC):
    h = jnp.maximum(
        dis * acc_ref[p].astype(jnp.float32) + b1_ref[p], 0.0)
    s_scr[p] += jnp.sum(wv * h, axis=0, keepdims=True)

  @pl.when(i == (N // BMD) - 1)
  def _():
    acc = jnp.zeros((1, D_OUT), jnp.float32)
    for p in range(NC):
      acc = acc + jnp.dot(s_scr[p], w2_ref[p],
                          preferred_element_type=jnp.float32)
    out_ref[...] = acc * (1.0 / N) + b2_ref[...]


def _tc_reduce(acc2, dis2, w2col, b1r, W2r, b2r):
  return pl.pallas_call(
      _red_body,
      out_shape=jax.ShapeDtypeStruct((1, D_OUT), jnp.float32),
      grid=(N // BMD,),
      in_specs=[
          pl.BlockSpec((NC, BMD, SLAB), lambda i: (0, i, 0)),
          pl.BlockSpec((BMD, 1), lambda i: (i, 0)),
          pl.BlockSpec((BMD, 1), lambda i: (i, 0)),
          pl.BlockSpec((NC, 1, SLAB), lambda i: (0, 0, 0)),
          pl.BlockSpec((NC, SLAB, D_OUT), lambda i: (0, 0, 0)),
          pl.BlockSpec((1, D_OUT), lambda i: (0, 0)),
      ],
      out_specs=pl.BlockSpec((1, D_OUT), lambda i: (0, 0)),
      scratch_shapes=[pltpu.VMEM((NC, 1, SLAB), jnp.float32)],
  )(acc2, dis2, w2col, b1r, W2r, b2r)


def kernel(x, edge_index, W1, b1, W2, b2):
  src = edge_index[0]
  dst = edge_index[1]

  dis_pad, w_pad = _sc_stats(src, dst)
  dis = dis_pad[:N]

  y = _tc_scaled_mm(x, W1, dis.reshape(N, 1))
  y2 = y.reshape(N * NC, SLAB)

  loop = jnp.arange(N, dtype=jnp.int32)
  pad = E2 - (E + N)
  src2 = jnp.concatenate([src, loop, jnp.zeros((pad,), jnp.int32)])
  dst2 = jnp.concatenate([dst, loop, jnp.full((pad,), N, jnp.int32)])
  gidx2 = (src2[None, :] * NC
           + jnp.arange(NC, dtype=jnp.int32)[:, None]).reshape(
               NC, NS * NB, BATCH)
  didx = dst2.reshape(NS * NB, BATCH)

  acc2 = _sc_message(y2, gidx2, didx)

  out = _tc_reduce(acc2,
                   dis.reshape(N, 1),
                   w_pad[:N].reshape(N, 1),
                   b1.reshape(NC, 1, SLAB),
                   W2.reshape(NC, SLAB, D_OUT),
                   b2.reshape(1, D_OUT))
  return out.reshape(D_OUT)
